# baseline (speedup 1.0000x reference)
import functools

import jax
import jax.numpy as jnp
from jax import lax
from jax.experimental import pallas as pl
from jax.experimental.pallas import tpu as pltpu
from jax.experimental.pallas import tpu_sc as plsc

NUM_ITEMS = 1000000
LATENT_DIM = 16
BATCH = 16384

NC = 2
NS = 16
NW = NC * NS
B_PER_W = BATCH // NW
CHUNK = 128
NCHUNK = B_PER_W // CHUNK
BLOCKS = B_PER_W // LATENT_DIM

_mesh = plsc.VectorSubcoreMesh(
    core_axis_name="c", subcore_axis_name="s", num_cores=NC, num_subcores=NS
)


@functools.partial(
    pl.kernel,
    out_type=jax.ShapeDtypeStruct((BATCH,), jnp.float32),
    mesh=_mesh,
    compiler_params=pltpu.CompilerParams(
        needs_layout_passes=False, use_tc_tiling_on_sc=False
    ),
    scratch_types=[
        pltpu.VMEM((NCHUNK, CHUNK), jnp.int32),
        pltpu.VMEM((B_PER_W, LATENT_DIM), jnp.float32),
        pltpu.VMEM((2 * LATENT_DIM,), jnp.float32),
        pltpu.VMEM((B_PER_W,), jnp.float32),
        pltpu.SemaphoreType.DMA,
    ],
)
def _sc_kernel(idx_hbm, table_hbm, params_hbm, out_hbm,
               idx_v, rows_v, params_v, out_v, sem):
    wid = lax.axis_index("s") * NC + lax.axis_index("c")
    base = wid * B_PER_W

    pltpu.sync_copy(idx_hbm.at[pl.ds(wid * NCHUNK, NCHUNK)], idx_v)
    pltpu.sync_copy(params_hbm, params_v)

    copies = [
        pltpu.make_async_copy(
            table_hbm.at[idx_v.at[c]],
            rows_v.at[pl.ds(c * CHUNK, CHUNK)],
            sem,
        )
        for c in range(NCHUNK)
    ]
    for cp in copies:
        cp.start()
    for cp in copies:
        cp.wait()

    w_cols = [
        plsc.load_gather(params_v, [jnp.full((16,), LATENT_DIM + d, jnp.int32)])
        for d in range(LATENT_DIM)
    ]
    b_vec = params_v[pl.ds(0, 16)]

    lane = lax.iota(jnp.int32, 16)

    def block_body(j, carry):
        row0 = j * LATENT_DIM
        rj = row0 + lane
        acc = b_vec
        for d in range(LATENT_DIM):
            col = plsc.load_gather(rows_v, [rj, jnp.full((16,), d, jnp.int32)])
            acc = acc + col * w_cols[d]
        out_v[pl.ds(row0, 16)] = 1.0 / (1.0 + jnp.exp(-acc))
        return carry

    lax.fori_loop(0, BLOCKS, block_body, 0)

    pltpu.sync_copy(out_v, out_hbm.at[pl.ds(base, B_PER_W)])


def kernel(item_indices, embedding_item, affine_W, affine_b):
    idx2 = item_indices.astype(jnp.int32).reshape(NW * NCHUNK, CHUNK)
    params = jnp.concatenate(
        [jnp.broadcast_to(affine_b.reshape(1), (LATENT_DIM,)),
         affine_W.reshape(LATENT_DIM)]
    )
    out = _sc_kernel(idx2, embedding_item, params)
    return out.reshape(BATCH, 1)


# speedup vs baseline: 9.2616x; 9.2616x over previous
import functools

import jax
import jax.numpy as jnp
from jax import lax
from jax.experimental import pallas as pl
from jax.experimental.pallas import tpu as pltpu
from jax.experimental.pallas import tpu_sc as plsc

NUM_ITEMS = 1000000
LATENT_DIM = 16
BATCH = 16384

NC = 2
NS = 16
NW = NC * NS
B_PER_W = BATCH // NW
CHUNK = 128
NCHUNK = B_PER_W // CHUNK
BLOCKS = B_PER_W // LATENT_DIM

TC_BLK = 65536
TC_GRID = (NUM_ITEMS + TC_BLK - 1) // TC_BLK


def _tc_body(t_ref, w_ref, y_ref):
    t = t_ref[...]
    w = w_ref[...][:, 0:1]
    y_ref[...] = jnp.sum(t * w, axis=0)


_tc_matvec = pl.pallas_call(
    _tc_body,
    out_shape=jax.ShapeDtypeStruct((NUM_ITEMS,), jnp.float32),
    grid=(TC_GRID,),
    in_specs=[
        pl.BlockSpec((LATENT_DIM, TC_BLK), lambda g: (0, g)),
        pl.BlockSpec((LATENT_DIM, 128), lambda g: (0, 0)),
    ],
    out_specs=pl.BlockSpec((TC_BLK,), lambda g: (g,)),
)

_mesh = plsc.VectorSubcoreMesh(
    core_axis_name="c", subcore_axis_name="s", num_cores=NC, num_subcores=NS
)


@functools.partial(
    pl.kernel,
    out_type=jax.ShapeDtypeStruct((BATCH,), jnp.float32),
    mesh=_mesh,
    compiler_params=pltpu.CompilerParams(
        needs_layout_passes=False, use_tc_tiling_on_sc=False
    ),
    scratch_types=[
        pltpu.VMEM((NCHUNK, CHUNK), jnp.int32),
        pltpu.VMEM((B_PER_W,), jnp.float32),
        pltpu.VMEM((LATENT_DIM,), jnp.float32),
        pltpu.VMEM((B_PER_W,), jnp.float32),
        pltpu.SemaphoreType.DMA,
    ],
)
def _sc_gather(idx_hbm, y_hbm, b_hbm, out_hbm,
               idx_v, g_v, b_v, out_v, sem):
    wid = lax.axis_index("s") * NC + lax.axis_index("c")
    base = wid * B_PER_W

    pltpu.sync_copy(idx_hbm.at[pl.ds(wid * NCHUNK, NCHUNK)], idx_v)
    pltpu.sync_copy(b_hbm, b_v)

    copies = [
        pltpu.make_async_copy(
            y_hbm.at[idx_v.at[c]],
            g_v.at[pl.ds(c * CHUNK, CHUNK)],
            sem,
        )
        for c in range(NCHUNK)
    ]
    for cp in copies:
        cp.start()
    for cp in copies:
        cp.wait()

    b_vec = b_v[...]

    def body(j, carry):
        i0 = j * 16
        acc = g_v[pl.ds(i0, 16)] + b_vec
        out_v[pl.ds(i0, 16)] = 1.0 / (1.0 + jnp.exp(-acc))
        return carry

    lax.fori_loop(0, BLOCKS, body, 0)

    pltpu.sync_copy(out_v, out_hbm.at[pl.ds(base, B_PER_W)])


def kernel(item_indices, embedding_item, affine_W, affine_b):
    idx2 = item_indices.astype(jnp.int32).reshape(NW * NCHUNK, CHUNK)
    w128 = jnp.broadcast_to(affine_W, (LATENT_DIM, 128))
    y = _tc_matvec(embedding_item.T, w128)
    b16 = jnp.broadcast_to(affine_b.reshape(1), (LATENT_DIM,))
    out = _sc_gather(idx2, y, b16)
    return out.reshape(BATCH, 1)


# speedup vs baseline: 9.9540x; 1.0748x over previous
import functools

import jax
import jax.numpy as jnp
from jax import lax
from jax.experimental import pallas as pl
from jax.experimental.pallas import tpu as pltpu
from jax.experimental.pallas import tpu_sc as plsc

NUM_ITEMS = 1000000
LATENT_DIM = 16
BATCH = 16384

NC = 2
NS = 16
NW = NC * NS
B_PER_W = BATCH // NW
CHUNK = 128
NCHUNK = B_PER_W // CHUNK
BLOCKS = B_PER_W // LATENT_DIM

TC_BLK = 131072
TC_GRID = (NUM_ITEMS + TC_BLK - 1) // TC_BLK


def _tc_body(t_ref, w_ref, y_ref):
    t = t_ref[...]
    w = w_ref[...][:, 0:1]
    y_ref[...] = jnp.sum(t * w, axis=0)


_tc_matvec = pl.pallas_call(
    _tc_body,
    out_shape=jax.ShapeDtypeStruct((NUM_ITEMS,), jnp.float32),
    grid=(TC_GRID,),
    in_specs=[
        pl.BlockSpec((LATENT_DIM, TC_BLK), lambda g: (0, g)),
        pl.BlockSpec((LATENT_DIM, 128), lambda g: (0, 0)),
    ],
    out_specs=pl.BlockSpec((TC_BLK,), lambda g: (g,)),
)

_mesh = plsc.VectorSubcoreMesh(
    core_axis_name="c", subcore_axis_name="s", num_cores=NC, num_subcores=NS
)


@functools.partial(
    pl.kernel,
    out_type=jax.ShapeDtypeStruct((BATCH,), jnp.float32),
    mesh=_mesh,
    compiler_params=pltpu.CompilerParams(
        needs_layout_passes=False, use_tc_tiling_on_sc=False
    ),
    scratch_types=[
        pltpu.VMEM((NCHUNK, CHUNK), jnp.int32),
        pltpu.VMEM((B_PER_W,), jnp.float32),
        pltpu.VMEM((LATENT_DIM,), jnp.float32),
        pltpu.VMEM((B_PER_W,), jnp.float32),
        pltpu.SemaphoreType.DMA,
    ],
)
def _sc_gather(idx_hbm, y_hbm, b_hbm, out_hbm,
               idx_v, g_v, b_v, out_v, sem):
    wid = lax.axis_index("s") * NC + lax.axis_index("c")
    base = wid * B_PER_W

    pltpu.sync_copy(idx_hbm.at[pl.ds(wid * NCHUNK, NCHUNK)], idx_v)
    pltpu.sync_copy(b_hbm, b_v)

    copies = [
        pltpu.make_async_copy(
            y_hbm.at[idx_v.at[c]],
            g_v.at[pl.ds(c * CHUNK, CHUNK)],
            sem,
        )
        for c in range(NCHUNK)
    ]
    for cp in copies:
        cp.start()
    for cp in copies:
        cp.wait()

    b_vec = b_v[...]

    def body(j, carry):
        i0 = j * 16
        acc = g_v[pl.ds(i0, 16)] + b_vec
        out_v[pl.ds(i0, 16)] = 1.0 / (1.0 + jnp.exp(-acc))
        return carry

    lax.fori_loop(0, BLOCKS, body, 0)

    pltpu.sync_copy(out_v, out_hbm.at[pl.ds(base, B_PER_W)])


def kernel(item_indices, embedding_item, affine_W, affine_b):
    idx2 = item_indices.astype(jnp.int32).reshape(NW * NCHUNK, CHUNK)
    w128 = jnp.broadcast_to(affine_W, (LATENT_DIM, 128))
    y = _tc_matvec(embedding_item.T, w128)
    b16 = jnp.broadcast_to(affine_b.reshape(1), (LATENT_DIM,))
    out = _sc_gather(idx2, y, b16)
    return out.reshape(BATCH, 1)


# speedup vs baseline: 9.9574x; 1.0003x over previous
import functools

import jax
import jax.numpy as jnp
from jax import lax
from jax.experimental import pallas as pl
from jax.experimental.pallas import tpu as pltpu
from jax.experimental.pallas import tpu_sc as plsc

NUM_ITEMS = 1000000
LATENT_DIM = 16
BATCH = 16384

NC = 2
NS = 16
NW = NC * NS
B_PER_W = BATCH // NW
CHUNK = 128
NCHUNK = B_PER_W // CHUNK
BLOCKS = B_PER_W // LATENT_DIM

TC_BLK = 262144
TC_GRID = (NUM_ITEMS + TC_BLK - 1) // TC_BLK


def _tc_body(t_ref, w_ref, y_ref):
    t = t_ref[...]
    w = w_ref[...][:, 0:1]
    y_ref[...] = jnp.sum(t * w, axis=0)


_tc_matvec = pl.pallas_call(
    _tc_body,
    out_shape=jax.ShapeDtypeStruct((NUM_ITEMS,), jnp.float32),
    grid=(TC_GRID,),
    in_specs=[
        pl.BlockSpec((LATENT_DIM, TC_BLK), lambda g: (0, g)),
        pl.BlockSpec((LATENT_DIM, 128), lambda g: (0, 0)),
    ],
    out_specs=pl.BlockSpec((TC_BLK,), lambda g: (g,)),
)

_mesh = plsc.VectorSubcoreMesh(
    core_axis_name="c", subcore_axis_name="s", num_cores=NC, num_subcores=NS
)


@functools.partial(
    pl.kernel,
    out_type=jax.ShapeDtypeStruct((BATCH,), jnp.float32),
    mesh=_mesh,
    compiler_params=pltpu.CompilerParams(
        needs_layout_passes=False, use_tc_tiling_on_sc=False
    ),
    scratch_types=[
        pltpu.VMEM((NCHUNK, CHUNK), jnp.int32),
        pltpu.VMEM((B_PER_W,), jnp.float32),
        pltpu.VMEM((LATENT_DIM,), jnp.float32),
        pltpu.VMEM((B_PER_W,), jnp.float32),
        pltpu.SemaphoreType.DMA,
    ],
)
def _sc_gather(idx_hbm, y_hbm, b_hbm, out_hbm,
               idx_v, g_v, b_v, out_v, sem):
    wid = lax.axis_index("s") * NC + lax.axis_index("c")
    base = wid * B_PER_W

    pltpu.sync_copy(idx_hbm.at[pl.ds(wid * NCHUNK, NCHUNK)], idx_v)
    pltpu.sync_copy(b_hbm, b_v)

    copies = [
        pltpu.make_async_copy(
            y_hbm.at[idx_v.at[c]],
            g_v.at[pl.ds(c * CHUNK, CHUNK)],
            sem,
        )
        for c in range(NCHUNK)
    ]
    for cp in copies:
        cp.start()
    for cp in copies:
        cp.wait()

    b_vec = b_v[...]

    def body(j, carry):
        i0 = j * 16
        acc = g_v[pl.ds(i0, 16)] + b_vec
        out_v[pl.ds(i0, 16)] = 1.0 / (1.0 + jnp.exp(-acc))
        return carry

    lax.fori_loop(0, BLOCKS, body, 0)

    pltpu.sync_copy(out_v, out_hbm.at[pl.ds(base, B_PER_W)])


def kernel(item_indices, embedding_item, affine_W, affine_b):
    idx2 = item_indices.astype(jnp.int32).reshape(NW * NCHUNK, CHUNK)
    w128 = jnp.broadcast_to(affine_W, (LATENT_DIM, 128))
    y = _tc_matvec(embedding_item.T, w128)
    b16 = jnp.broadcast_to(affine_b.reshape(1), (LATENT_DIM,))
    out = _sc_gather(idx2, y, b16)
    return out.reshape(BATCH, 1)
